# Initial kernel scaffold; baseline (speedup 1.0000x reference)
import functools

import jax
import jax.numpy as jnp
from jax import lax
from jax.experimental import pallas as pl
from jax.experimental.pallas import tpu as pltpu
from jax.experimental.pallas import tpu_sc as plsc

N_NODES = 10000
N_EDGES = 320000
DIM = 128

NC = 2
NS = 16
NW = NC * NS
E_PER_W = N_EDGES // NW
CHUNK = 80
N_CHUNKS = E_PER_W // CHUNK
ROWS_PER_TILE = N_NODES // NS
ZROWS = 125


def _mm_body(x_ref, w_ref, o_ref):
    o_ref[...] = lax.dot_general(
        x_ref[...], w_ref[...], (((1,), (1,)), ((), ())),
        preferred_element_type=jnp.float32)


def _linear(node_emb, W):
    grid = 10
    bm = N_NODES // grid
    return pl.pallas_call(
        _mm_body,
        grid=(grid,),
        in_specs=[
            pl.BlockSpec((bm, DIM), lambda i: (i, 0)),
            pl.BlockSpec((DIM, DIM), lambda i: (0, 0)),
        ],
        out_specs=pl.BlockSpec((bm, DIM), lambda i: (i, 0)),
        out_shape=jax.ShapeDtypeStruct((N_NODES, DIM), jnp.float32),
    )(node_emb, W)


def _add_body(a_ref, b_ref, o_ref):
    o_ref[...] = a_ref[...] + b_ref[...]


def _sum_partials(p0, p1):
    grid = 10
    bm = N_NODES // grid
    return pl.pallas_call(
        _add_body,
        grid=(grid,),
        in_specs=[
            pl.BlockSpec((bm, DIM), lambda i: (i, 0)),
            pl.BlockSpec((bm, DIM), lambda i: (i, 0)),
        ],
        out_specs=pl.BlockSpec((bm, DIM), lambda i: (i, 0)),
        out_shape=jax.ShapeDtypeStruct((N_NODES, DIM), jnp.float32),
    )(p0, p1)


def _sc_body(h_hbm, src_hbm, dst_hbm, w_hbm, out_hbm,
             acc, src_v, dst_v, w_v, rows_v, stage_v, sem):
    cid = lax.axis_index("c")
    sid = lax.axis_index("s")
    wid = cid * NS + sid

    zeros16 = jnp.zeros((16,), jnp.float32)

    @pl.loop(0, ZROWS)
    def _zero(i):
        for j in range(DIM // 16):
            stage_v[i, pl.ds(j * 16, 16)] = zeros16

    for k in range(ROWS_PER_TILE // ZROWS):
        pltpu.sync_copy(stage_v, acc.at[pl.ds(sid * ROWS_PER_TILE + k * ZROWS, ZROWS)])

    plsc.subcore_barrier()

    @pl.loop(0, N_CHUNKS)
    def _chunk(c):
        base = wid * E_PER_W + c * CHUNK
        pltpu.sync_copy(src_hbm.at[pl.ds(base, CHUNK)], src_v)
        pltpu.sync_copy(dst_hbm.at[pl.ds(base, CHUNK)], dst_v)
        pltpu.sync_copy(w_hbm.at[pl.ds(base, CHUNK)], w_v)
        pltpu.async_copy(h_hbm.at[src_v], rows_v, sem).wait()

        @pl.loop(0, CHUNK)
        def _scale(i):
            wt = w_v[i]
            for j in range(DIM // 16):
                sl = pl.ds(j * 16, 16)
                rows_v[i, sl] = rows_v[i, sl] * wt

        pltpu.sync_copy(rows_v, acc.at[dst_v], add=True)

    plsc.subcore_barrier()

    for k in range(ROWS_PER_TILE // ZROWS):
        r0 = sid * ROWS_PER_TILE + k * ZROWS
        pltpu.sync_copy(acc.at[pl.ds(r0, ZROWS)], stage_v)
        pltpu.sync_copy(stage_v, out_hbm.at[pl.ds(cid * N_NODES + r0, ZROWS)])


_sc_edges = pl.kernel(
    _sc_body,
    out_type=jax.ShapeDtypeStruct((NC * N_NODES, DIM), jnp.float32),
    mesh=plsc.VectorSubcoreMesh(core_axis_name="c", subcore_axis_name="s"),
    scratch_types=[
        pltpu.VMEM_SHARED((N_NODES, DIM), jnp.float32),
        pltpu.VMEM((CHUNK,), jnp.int32),
        pltpu.VMEM((CHUNK,), jnp.int32),
        pltpu.VMEM((CHUNK,), jnp.float32),
        pltpu.VMEM((CHUNK, DIM), jnp.float32),
        pltpu.VMEM((ZROWS, DIM), jnp.float32),
        pltpu.SemaphoreType.DMA,
    ],
)


def kernel(node_emb, edges, edge_weight, W):
    dst = edges[0].astype(jnp.int32)
    src = edges[1].astype(jnp.int32)
    h = _linear(node_emb, W)
    partials = _sc_edges(h, src, dst, edge_weight)
    return _sum_partials(partials[:N_NODES], partials[N_NODES:])


# speedup vs baseline: 4.0669x; 4.0669x over previous
import functools

import jax
import jax.numpy as jnp
from jax import lax
from jax.experimental import pallas as pl
from jax.experimental.pallas import tpu as pltpu
from jax.experimental.pallas import tpu_sc as plsc

N_NODES = 10000
N_EDGES = 320000
DIM = 128

NC = 2
NS = 16
NW = NC * NS
E_PER_W = N_EDGES // NW
CHUNK = 80
N_CHUNKS = E_PER_W // CHUNK
N_PAD = 10240
ROWS_PER_TILE = N_PAD // NS
ZROWS = 128


def _mm_body(x_ref, w_ref, o_ref):
    o_ref[...] = lax.dot_general(
        x_ref[...], w_ref[...], (((1,), (1,)), ((), ())),
        preferred_element_type=jnp.float32)


def _linear(node_emb, W):
    grid = 10
    bm = N_NODES // grid
    return pl.pallas_call(
        _mm_body,
        grid=(grid,),
        in_specs=[
            pl.BlockSpec((bm, DIM), lambda i: (i, 0)),
            pl.BlockSpec((DIM, DIM), lambda i: (0, 0)),
        ],
        out_specs=pl.BlockSpec((bm, DIM), lambda i: (i, 0)),
        out_shape=jax.ShapeDtypeStruct((N_NODES, DIM), jnp.float32),
    )(node_emb, W)


def _add_body(a_ref, b_ref, o_ref):
    o_ref[...] = a_ref[...] + b_ref[...]


def _sum_partials(partials):
    grid = 10
    bm = N_PAD // grid
    return pl.pallas_call(
        _add_body,
        grid=(grid,),
        in_specs=[
            pl.BlockSpec((bm, DIM), lambda i: (i, 0)),
            pl.BlockSpec((bm, DIM), lambda i: (i + grid, 0)),
        ],
        out_specs=pl.BlockSpec((bm, DIM), lambda i: (i, 0)),
        out_shape=jax.ShapeDtypeStruct((N_PAD, DIM), jnp.float32),
    )(partials, partials)


def _sc_body(h_hbm, src_hbm, dst_hbm, w_hbm, out_hbm,
             acc, src_v, dst_v, w_v, rows_v, stage_v, sem):
    cid = lax.axis_index("c")
    sid = lax.axis_index("s")
    wid = cid * NS + sid

    zeros16 = jnp.zeros((16,), jnp.float32)

    @pl.loop(0, ZROWS)
    def _zero(i):
        for j in range(DIM // 16):
            stage_v[i, pl.ds(j * 16, 16)] = zeros16

    for k in range(ROWS_PER_TILE // ZROWS):
        pltpu.sync_copy(stage_v, acc.at[pl.ds(sid * ROWS_PER_TILE + k * ZROWS, ZROWS)])

    plsc.subcore_barrier()

    @pl.loop(0, N_CHUNKS)
    def _chunk(c):
        base = wid * E_PER_W + c * CHUNK
        pltpu.sync_copy(src_hbm.at[pl.ds(base, CHUNK)], src_v)
        pltpu.sync_copy(dst_hbm.at[pl.ds(base, CHUNK)], dst_v)
        pltpu.sync_copy(w_hbm.at[pl.ds(base, CHUNK)], w_v)
        pltpu.async_copy(h_hbm.at[src_v], rows_v, sem).wait()

        @pl.loop(0, CHUNK // 16)
        def _scale(g):
            w16 = w_v[pl.ds(g * 16, 16)]
            for e in range(16):
                wt = w16[e]
                r = g * 16 + e
                for j in range(DIM // 16):
                    sl = pl.ds(j * 16, 16)
                    rows_v[r, sl] = rows_v[r, sl] * wt

        pltpu.sync_copy(rows_v, acc.at[dst_v], add=True)

    plsc.subcore_barrier()

    for k in range(ROWS_PER_TILE // ZROWS):
        r0 = sid * ROWS_PER_TILE + k * ZROWS
        pltpu.sync_copy(acc.at[pl.ds(r0, ZROWS)], stage_v)
        pltpu.sync_copy(stage_v, out_hbm.at[pl.ds(cid * N_PAD + r0, ZROWS)])


_sc_edges = pl.kernel(
    _sc_body,
    out_type=jax.ShapeDtypeStruct((NC * N_PAD, DIM), jnp.float32),
    mesh=plsc.VectorSubcoreMesh(core_axis_name="c", subcore_axis_name="s"),
    scratch_types=[
        pltpu.VMEM_SHARED((N_PAD, DIM), jnp.float32),
        pltpu.VMEM((CHUNK,), jnp.int32),
        pltpu.VMEM((CHUNK,), jnp.int32),
        pltpu.VMEM((CHUNK,), jnp.float32),
        pltpu.VMEM((CHUNK, DIM), jnp.float32),
        pltpu.VMEM((ZROWS, DIM), jnp.float32),
        pltpu.SemaphoreType.DMA,
    ],
)


def kernel(node_emb, edges, edge_weight, W):
    dst = edges[0].astype(jnp.int32)
    src = edges[1].astype(jnp.int32)
    h = _linear(node_emb, W)
    partials = _sc_edges(h, src, dst, edge_weight)
    return _sum_partials(partials)[:N_NODES]


# speedup vs baseline: 8.9256x; 2.1947x over previous
import jax
import jax.numpy as jnp
from jax import lax
from jax.experimental import pallas as pl
from jax.experimental.pallas import tpu as pltpu
from jax.experimental.pallas import tpu_sc as plsc

N_NODES = 10000
N_EDGES = 320000
DIM = 128

NC = 2
NS = 16
NW = NC * NS
E_PER_W = N_EDGES // NW
CHUNK = 80
N_CHUNKS = E_PER_W // CHUNK
N_PAD = 10240
ROWS_PER_TILE = N_PAD // NS


def _mm_body(x_ref, w_ref, o_ref):
    o_ref[...] = lax.dot_general(
        x_ref[...], w_ref[...], (((1,), (1,)), ((), ())),
        preferred_element_type=jnp.float32)


def _linear(node_emb, W):
    grid = 10
    bm = N_NODES // grid
    return pl.pallas_call(
        _mm_body,
        grid=(grid,),
        in_specs=[
            pl.BlockSpec((bm, DIM), lambda i: (i, 0)),
            pl.BlockSpec((DIM, DIM), lambda i: (0, 0)),
        ],
        out_specs=pl.BlockSpec((bm, DIM), lambda i: (i, 0)),
        out_shape=jax.ShapeDtypeStruct((N_NODES, DIM), jnp.float32),
    )(node_emb, W)


def _add_body(a_ref, b_ref, o_ref):
    o_ref[...] = a_ref[...] + b_ref[...]


def _sum_partials(partials):
    grid = 10
    bm = N_PAD // grid
    return pl.pallas_call(
        _add_body,
        grid=(grid,),
        in_specs=[
            pl.BlockSpec((bm, DIM), lambda i: (i, 0)),
            pl.BlockSpec((bm, DIM), lambda i: (i + grid, 0)),
        ],
        out_specs=pl.BlockSpec((bm, DIM), lambda i: (i, 0)),
        out_shape=jax.ShapeDtypeStruct((N_PAD, DIM), jnp.float32),
    )(partials, partials)


def _sc_body(h_hbm, src_hbm, dst_hbm, w_hbm, out_hbm,
             acc, src_all, w_all, dstb0, dstb1, rows0, rows1,
             gsem0, gsem1, ssem0, ssem1, dsem0, dsem1):
    cid = lax.axis_index("c")
    sid = lax.axis_index("s")
    wid = cid * NS + sid
    ebase = wid * E_PER_W

    rows = (rows0, rows1)
    dstb = (dstb0, dstb1)
    gsem = (gsem0, gsem1)
    ssem = (ssem0, ssem1)
    dsem = (dsem0, dsem1)

    pltpu.sync_copy(src_hbm.at[pl.ds(ebase, E_PER_W)], src_all)
    pltpu.sync_copy(w_hbm.at[pl.ds(ebase, E_PER_W)], w_all)

    zeros16 = jnp.zeros((16,), jnp.float32)

    @pl.loop(0, CHUNK)
    def _zero(i):
        for j in range(DIM // 16):
            rows0[i, pl.ds(j * 16, 16)] = zeros16

    for k in range(ROWS_PER_TILE // CHUNK):
        pltpu.sync_copy(rows0, acc.at[pl.ds(sid * ROWS_PER_TILE + k * CHUNK, CHUNK)])

    plsc.subcore_barrier()

    def launch_gather(c, p):
        pltpu.async_copy(h_hbm.at[src_all.at[pl.ds(c * CHUNK, CHUNK)]], rows[p],
                         gsem[p])

    def launch_dst(c, p):
        pltpu.async_copy(dst_hbm.at[pl.ds(ebase + c * CHUNK, CHUNK)], dstb[p],
                         dsem[p])

    def wait_gather(p):
        pltpu.make_async_copy(h_hbm.at[pl.ds(0, CHUNK)], rows[p], gsem[p]).wait()

    def wait_dst(p):
        pltpu.make_async_copy(dst_hbm.at[pl.ds(0, CHUNK)], dstb[p], dsem[p]).wait()

    def launch_scatter(c, p):
        pltpu.async_copy(rows[p], acc.at[dstb[p]], ssem[p], add=True)

    def wait_scatter(p):
        pltpu.make_async_copy(rows[p], acc.at[dstb[p]], ssem[p]).wait()

    def scale(c, p):
        @pl.loop(0, CHUNK // 16)
        def _scale(g):
            w16 = w_all[pl.ds(c * CHUNK + g * 16, 16)]
            for e in range(16):
                wt = w16[e]
                r = g * 16 + e
                for j in range(DIM // 16):
                    sl = pl.ds(j * 16, 16)
                    rows[p][r, sl] = rows[p][r, sl] * wt

    def step(c, p, first, last):
        wait_gather(p)
        if not first:
            wait_scatter(1 - p)
        if not last:
            launch_dst(c + 1, 1 - p)
            launch_gather(c + 1, 1 - p)
        scale(c, p)
        wait_dst(p)
        launch_scatter(c, p)

    launch_dst(0, 0)
    launch_gather(0, 0)

    step(0, 0, first=True, last=False)
    step(1, 1, first=False, last=False)

    @pl.loop(0, (N_CHUNKS - 3) // 2)
    def _pair(c2):
        c = 2 + 2 * c2
        step(c, 0, first=False, last=False)
        step(c + 1, 1, first=False, last=False)

    step(N_CHUNKS - 1, (N_CHUNKS - 1) % 2, first=False, last=True)

    wait_scatter((N_CHUNKS - 1) % 2)

    plsc.subcore_barrier()

    r0 = sid * ROWS_PER_TILE
    pltpu.sync_copy(acc.at[pl.ds(r0, ROWS_PER_TILE)],
                    out_hbm.at[pl.ds(cid * N_PAD + r0, ROWS_PER_TILE)])


_sc_edges = pl.kernel(
    _sc_body,
    out_type=jax.ShapeDtypeStruct((NC * N_PAD, DIM), jnp.float32),
    mesh=plsc.VectorSubcoreMesh(core_axis_name="c", subcore_axis_name="s"),
    scratch_types=[
        pltpu.VMEM_SHARED((N_PAD, DIM), jnp.float32),
        pltpu.VMEM((E_PER_W,), jnp.int32),
        pltpu.VMEM((E_PER_W,), jnp.float32),
        pltpu.VMEM((CHUNK,), jnp.int32),
        pltpu.VMEM((CHUNK,), jnp.int32),
        pltpu.VMEM((CHUNK, DIM), jnp.float32),
        pltpu.VMEM((CHUNK, DIM), jnp.float32),
        pltpu.SemaphoreType.DMA,
        pltpu.SemaphoreType.DMA,
        pltpu.SemaphoreType.DMA,
        pltpu.SemaphoreType.DMA,
        pltpu.SemaphoreType.DMA,
        pltpu.SemaphoreType.DMA,
    ],
)


def kernel(node_emb, edges, edge_weight, W):
    dst = edges[0].astype(jnp.int32)
    src = edges[1].astype(jnp.int32)
    h = _linear(node_emb, W)
    partials = _sc_edges(h, src, dst, edge_weight)
    return _sum_partials(partials)[:N_NODES]


# speedup vs baseline: 10.1163x; 1.1334x over previous
import jax
import jax.numpy as jnp
from jax import lax
from jax.experimental import pallas as pl
from jax.experimental.pallas import tpu as pltpu
from jax.experimental.pallas import tpu_sc as plsc

N_NODES = 10000
N_EDGES = 320000
DIM = 128

NC = 2
NS = 16
NW = NC * NS
E_PER_W = N_EDGES // NW
CHUNK = 80
N_CHUNKS = E_PER_W // CHUNK
N_PAD = 10240
ROWS_PER_TILE = N_PAD // NS
NBUF = 2


def _fused_body(a_ref, b_ref, w_ref, o_ref):
    o_ref[...] = lax.dot_general(
        a_ref[...] + b_ref[...], w_ref[...], (((1,), (1,)), ((), ())),
        preferred_element_type=jnp.float32)


def _sum_matmul(partials, W):
    grid = 10
    bm = N_PAD // grid
    return pl.pallas_call(
        _fused_body,
        grid=(grid,),
        in_specs=[
            pl.BlockSpec((bm, DIM), lambda i: (i, 0)),
            pl.BlockSpec((bm, DIM), lambda i: (i + grid, 0)),
            pl.BlockSpec((DIM, DIM), lambda i: (0, 0)),
        ],
        out_specs=pl.BlockSpec((bm, DIM), lambda i: (i, 0)),
        out_shape=jax.ShapeDtypeStruct((N_PAD, DIM), jnp.float32),
    )(partials, partials, W)


def _sc_body(x_hbm, src_hbm, dst_hbm, w_hbm, out_hbm,
             acc, src_all, w_all, *bufs):
    dstb = bufs[0:NBUF]
    rows = bufs[NBUF:2 * NBUF]
    gsem = bufs[2 * NBUF:3 * NBUF]
    ssem = bufs[3 * NBUF:4 * NBUF]
    dsem = bufs[4 * NBUF:5 * NBUF]

    cid = lax.axis_index("c")
    sid = lax.axis_index("s")
    wid = cid * NS + sid
    ebase = wid * E_PER_W

    pltpu.sync_copy(src_hbm.at[pl.ds(ebase, E_PER_W)], src_all)
    pltpu.sync_copy(w_hbm.at[pl.ds(ebase, E_PER_W)], w_all)

    zeros16 = jnp.zeros((16,), jnp.float32)

    @pl.loop(0, CHUNK)
    def _zero(i):
        for j in range(DIM // 16):
            rows[0][i, pl.ds(j * 16, 16)] = zeros16

    for k in range(ROWS_PER_TILE // CHUNK):
        pltpu.sync_copy(rows[0], acc.at[pl.ds(sid * ROWS_PER_TILE + k * CHUNK, CHUNK)])

    plsc.subcore_barrier()

    def launch(c, p):
        pltpu.async_copy(dst_hbm.at[pl.ds(ebase + c * CHUNK, CHUNK)], dstb[p],
                         dsem[p])
        pltpu.async_copy(x_hbm.at[src_all.at[pl.ds(c * CHUNK, CHUNK)]], rows[p],
                         gsem[p])

    def wait_gather(p):
        pltpu.make_async_copy(x_hbm.at[pl.ds(0, CHUNK)], rows[p], gsem[p]).wait()

    def wait_dst(p):
        pltpu.make_async_copy(dst_hbm.at[pl.ds(0, CHUNK)], dstb[p], dsem[p]).wait()

    def launch_scatter(c, p):
        pltpu.async_copy(rows[p], acc.at[dstb[p]], ssem[p], add=True)

    def wait_scatter(p):
        pltpu.make_async_copy(rows[p], acc.at[dstb[p]], ssem[p]).wait()

    def scale(c, p):
        @pl.loop(0, CHUNK // 16)
        def _scale(g):
            w16 = w_all[pl.ds(c * CHUNK + g * 16, 16)]
            for e in range(16):
                wt = w16[e]
                r = g * 16 + e
                for j in range(DIM // 16):
                    sl = pl.ds(j * 16, 16)
                    rows[p][r, sl] = rows[p][r, sl] * wt

    def step(c, p, guard):
        rb = (p - 1) % NBUF
        if guard == "static":
            wait_scatter(rb)
            launch(c + NBUF - 1, rb)
        elif guard == "traced":
            @pl.when(c < N_CHUNKS - (NBUF - 1))
            def _():
                wait_scatter(rb)
                launch(c + NBUF - 1, rb)
        wait_gather(p)
        scale(c, p)
        wait_dst(p)
        launch_scatter(c, p)

    for c in range(NBUF - 1):
        launch(c, c)

    launch(NBUF - 1, NBUF - 1)
    wait_gather(0)
    scale(0, 0)
    wait_dst(0)
    launch_scatter(0, 0)

    for c in range(1, NBUF - 1):
        step(c, c, "static")

    n_main = N_CHUNKS - (NBUF - 1)
    n_quads = n_main // NBUF
    base_p = (NBUF - 1) % NBUF

    @pl.loop(0, n_quads)
    def _quad(q):
        c0 = (NBUF - 1) + q * NBUF
        for k in range(NBUF):
            step(c0 + k, (base_p + k) % NBUF, "traced")

    for c in range((NBUF - 1) + n_quads * NBUF, N_CHUNKS):
        step(c, c % NBUF, "traced-tail")

    for c in range(N_CHUNKS - NBUF, N_CHUNKS):
        wait_scatter(c % NBUF)

    plsc.subcore_barrier()

    r0 = sid * ROWS_PER_TILE
    pltpu.sync_copy(acc.at[pl.ds(r0, ROWS_PER_TILE)],
                    out_hbm.at[pl.ds(cid * N_PAD + r0, ROWS_PER_TILE)])


_sc_edges = pl.kernel(
    _sc_body,
    out_type=jax.ShapeDtypeStruct((NC * N_PAD, DIM), jnp.float32),
    mesh=plsc.VectorSubcoreMesh(core_axis_name="c", subcore_axis_name="s"),
    scratch_types=(
        [pltpu.VMEM_SHARED((N_PAD, DIM), jnp.float32),
         pltpu.VMEM((E_PER_W,), jnp.int32),
         pltpu.VMEM((E_PER_W,), jnp.float32)]
        + [pltpu.VMEM((CHUNK,), jnp.int32) for _ in range(NBUF)]
        + [pltpu.VMEM((CHUNK, DIM), jnp.float32) for _ in range(NBUF)]
        + [pltpu.SemaphoreType.DMA for _ in range(3 * NBUF)]
    ),
)


def kernel(node_emb, edges, edge_weight, W):
    dst = edges[0].astype(jnp.int32)
    src = edges[1].astype(jnp.int32)
    partials = _sc_edges(node_emb, src, dst, edge_weight)
    return _sum_matmul(partials, W)[:N_NODES]


# speedup vs baseline: 10.2202x; 1.0103x over previous
import jax
import jax.numpy as jnp
from jax import lax
from jax.experimental import pallas as pl
from jax.experimental.pallas import tpu as pltpu
from jax.experimental.pallas import tpu_sc as plsc

N_NODES = 10000
N_EDGES = 320000
DIM = 128

NC = 2
NS = 16
NW = NC * NS
E_PER_W = N_EDGES // NW
CHUNK = 80
N_CHUNKS = E_PER_W // CHUNK
N_PAD = 10240
ROWS_PER_TILE = N_PAD // NS
NBUF = 2


def _fused_body(a_ref, b_ref, w_ref, o_ref):
    o_ref[...] = lax.dot_general(
        a_ref[...] + b_ref[...], w_ref[...], (((1,), (1,)), ((), ())),
        preferred_element_type=jnp.float32)


def _sum_matmul(partials, W):
    grid = 10
    bm = N_PAD // grid
    return pl.pallas_call(
        _fused_body,
        grid=(grid,),
        in_specs=[
            pl.BlockSpec((bm, DIM), lambda i: (i, 0)),
            pl.BlockSpec((bm, DIM), lambda i: (i + grid, 0)),
            pl.BlockSpec((DIM, DIM), lambda i: (0, 0)),
        ],
        out_specs=pl.BlockSpec((bm, DIM), lambda i: (i, 0)),
        out_shape=jax.ShapeDtypeStruct((N_PAD, DIM), jnp.float32),
    )(partials, partials, W)


def _sc_body(x_hbm, src_hbm, dst_hbm, w_hbm, out_hbm,
             acc, src_all, w_all, *bufs):
    dstb = bufs[0:NBUF]
    rows = bufs[NBUF:2 * NBUF]
    gsem = bufs[2 * NBUF:3 * NBUF]
    ssem = bufs[3 * NBUF:4 * NBUF]
    dsem = bufs[4 * NBUF:5 * NBUF]

    cid = lax.axis_index("c")
    sid = lax.axis_index("s")
    wid = cid * NS + sid
    ebase = wid * E_PER_W

    pltpu.async_copy(src_hbm.at[pl.ds(ebase, E_PER_W)], src_all, gsem[0])
    pltpu.async_copy(w_hbm.at[pl.ds(ebase, E_PER_W)], w_all, gsem[1])

    zeros16 = jnp.zeros((16,), jnp.float32)

    @pl.loop(0, CHUNK)
    def _zero(i):
        for j in range(DIM // 16):
            rows[0][i, pl.ds(j * 16, 16)] = zeros16

    for k in range(ROWS_PER_TILE // CHUNK):
        pltpu.sync_copy(rows[0], acc.at[pl.ds(sid * ROWS_PER_TILE + k * CHUNK, CHUNK)])

    pltpu.make_async_copy(src_hbm.at[pl.ds(0, E_PER_W)], src_all, gsem[0]).wait()
    pltpu.make_async_copy(w_hbm.at[pl.ds(0, E_PER_W)], w_all, gsem[1]).wait()

    plsc.subcore_barrier()

    def launch(c, p):
        pltpu.async_copy(dst_hbm.at[pl.ds(ebase + c * CHUNK, CHUNK)], dstb[p],
                         dsem[p])
        pltpu.async_copy(x_hbm.at[src_all.at[pl.ds(c * CHUNK, CHUNK)]], rows[p],
                         gsem[p])

    def wait_gather(p):
        pltpu.make_async_copy(x_hbm.at[pl.ds(0, CHUNK)], rows[p], gsem[p]).wait()

    def wait_dst(p):
        pltpu.make_async_copy(dst_hbm.at[pl.ds(0, CHUNK)], dstb[p], dsem[p]).wait()

    def launch_scatter(c, p):
        pltpu.async_copy(rows[p], acc.at[dstb[p]], ssem[p], add=True)

    def wait_scatter(p):
        pltpu.make_async_copy(rows[p], acc.at[dstb[p]], ssem[p]).wait()

    def scale(c, p):
        @pl.loop(0, CHUNK // 16)
        def _scale(g):
            w16 = w_all[pl.ds(c * CHUNK + g * 16, 16)]
            for e in range(16):
                wt = w16[e]
                r = g * 16 + e
                for j in range(DIM // 16):
                    sl = pl.ds(j * 16, 16)
                    rows[p][r, sl] = rows[p][r, sl] * wt

    def step(c, p, guard):
        rb = (p - 1) % NBUF
        if guard == "static":
            wait_scatter(rb)
            launch(c + NBUF - 1, rb)
        elif guard == "traced":
            @pl.when(c < N_CHUNKS - (NBUF - 1))
            def _():
                wait_scatter(rb)
                launch(c + NBUF - 1, rb)
        wait_gather(p)
        scale(c, p)
        wait_dst(p)
        launch_scatter(c, p)

    for c in range(NBUF - 1):
        launch(c, c)

    launch(NBUF - 1, NBUF - 1)
    wait_gather(0)
    scale(0, 0)
    wait_dst(0)
    launch_scatter(0, 0)

    for c in range(1, NBUF - 1):
        step(c, c, "static")

    n_main = N_CHUNKS - (NBUF - 1)
    n_quads = n_main // NBUF
    base_p = (NBUF - 1) % NBUF

    @pl.loop(0, n_quads)
    def _quad(q):
        c0 = (NBUF - 1) + q * NBUF
        for k in range(NBUF):
            step(c0 + k, (base_p + k) % NBUF, "traced")

    for c in range((NBUF - 1) + n_quads * NBUF, N_CHUNKS):
        step(c, c % NBUF, "traced-tail")

    for c in range(N_CHUNKS - NBUF, N_CHUNKS):
        wait_scatter(c % NBUF)

    plsc.subcore_barrier()

    r0 = sid * ROWS_PER_TILE
    pltpu.sync_copy(acc.at[pl.ds(r0, ROWS_PER_TILE)],
                    out_hbm.at[pl.ds(cid * N_PAD + r0, ROWS_PER_TILE)])


_sc_edges = pl.kernel(
    _sc_body,
    out_type=jax.ShapeDtypeStruct((NC * N_PAD, DIM), jnp.float32),
    mesh=plsc.VectorSubcoreMesh(core_axis_name="c", subcore_axis_name="s"),
    scratch_types=(
        [pltpu.VMEM_SHARED((N_PAD, DIM), jnp.float32),
         pltpu.VMEM((E_PER_W,), jnp.int32),
         pltpu.VMEM((E_PER_W,), jnp.float32)]
        + [pltpu.VMEM((CHUNK,), jnp.int32) for _ in range(NBUF)]
        + [pltpu.VMEM((CHUNK, DIM), jnp.float32) for _ in range(NBUF)]
        + [pltpu.SemaphoreType.DMA for _ in range(3 * NBUF)]
    ),
)


def kernel(node_emb, edges, edge_weight, W):
    dst = edges[0].astype(jnp.int32)
    src = edges[1].astype(jnp.int32)
    partials = _sc_edges(node_emb, src, dst, edge_weight)
    return _sum_matmul(partials, W)[:N_NODES]


# speedup vs baseline: 10.2371x; 1.0016x over previous
import jax
import jax.numpy as jnp
from jax import lax
from jax.experimental import pallas as pl
from jax.experimental.pallas import tpu as pltpu
from jax.experimental.pallas import tpu_sc as plsc

N_NODES = 10000
N_EDGES = 320000
DIM = 128

NC = 2
NS = 16
NW = NC * NS
E_PER_W = N_EDGES // NW
CHUNK = 80
N_CHUNKS = E_PER_W // CHUNK
N_PAD = 10240
ROWS_PER_TILE = N_PAD // NS
NBUF = 2


def _fused_body(a_ref, b_ref, w_ref, o_ref):
    o_ref[...] = lax.dot_general(
        a_ref[...] + b_ref[...], w_ref[...], (((1,), (1,)), ((), ())),
        preferred_element_type=jnp.float32)


def _sum_matmul(partials, W):
    grid = 10
    bm = N_PAD // grid
    return pl.pallas_call(
        _fused_body,
        grid=(grid,),
        in_specs=[
            pl.BlockSpec((bm, DIM), lambda i: (i, 0)),
            pl.BlockSpec((bm, DIM), lambda i: (i + grid, 0)),
            pl.BlockSpec((DIM, DIM), lambda i: (0, 0)),
        ],
        out_specs=pl.BlockSpec((bm, DIM), lambda i: (i, 0)),
        out_shape=jax.ShapeDtypeStruct((N_PAD, DIM), jnp.float32),
    )(partials, partials, W)


def _sc_body(x_hbm, src_hbm, dst_hbm, w_hbm, out_hbm,
             acc, src_all, w_all, *bufs):
    dstb = bufs[0:NBUF]
    rows = bufs[NBUF:2 * NBUF]
    gsem = bufs[2 * NBUF:3 * NBUF]
    ssem = bufs[3 * NBUF:4 * NBUF]
    dsem = bufs[4 * NBUF:5 * NBUF]

    cid = lax.axis_index("c")
    sid = lax.axis_index("s")
    wid = cid * NS + sid
    ebase = wid * E_PER_W

    pltpu.async_copy(src_hbm.at[pl.ds(ebase, E_PER_W)], src_all, gsem[0])
    pltpu.async_copy(w_hbm.at[pl.ds(ebase, E_PER_W)], w_all, gsem[1])

    zeros16 = jnp.zeros((16,), jnp.float32)

    @pl.loop(0, CHUNK)
    def _zero(i):
        for j in range(DIM // 16):
            rows[0][i, pl.ds(j * 16, 16)] = zeros16

    for k in range(ROWS_PER_TILE // CHUNK):
        pltpu.sync_copy(rows[0], acc.at[pl.ds(sid * ROWS_PER_TILE + k * CHUNK, CHUNK)])

    pltpu.make_async_copy(src_hbm.at[pl.ds(0, E_PER_W)], src_all, gsem[0]).wait()
    pltpu.make_async_copy(w_hbm.at[pl.ds(0, E_PER_W)], w_all, gsem[1]).wait()

    plsc.subcore_barrier()

    def launch(c, p):
        pltpu.async_copy(dst_hbm.at[pl.ds(ebase + c * CHUNK, CHUNK)], dstb[p],
                         dsem[p])
        h = CHUNK // 2
        pltpu.async_copy(x_hbm.at[src_all.at[pl.ds(c * CHUNK, h)]],
                         rows[p].at[pl.ds(0, h)], gsem[p])
        pltpu.async_copy(x_hbm.at[src_all.at[pl.ds(c * CHUNK + h, h)]],
                         rows[p].at[pl.ds(h, h)], gsem[p])

    def wait_gather(p):
        pltpu.make_async_copy(x_hbm.at[pl.ds(0, CHUNK)], rows[p], gsem[p]).wait()

    def wait_dst(p):
        pltpu.make_async_copy(dst_hbm.at[pl.ds(0, CHUNK)], dstb[p], dsem[p]).wait()

    def launch_scatter(c, p):
        pltpu.async_copy(rows[p], acc.at[dstb[p]], ssem[p], add=True)

    def wait_scatter(p):
        pltpu.make_async_copy(rows[p], acc.at[dstb[p]], ssem[p]).wait()

    def scale(c, p):
        @pl.loop(0, CHUNK // 16)
        def _scale(g):
            w16 = w_all[pl.ds(c * CHUNK + g * 16, 16)]
            for e in range(16):
                wt = w16[e]
                r = g * 16 + e
                for j in range(DIM // 16):
                    sl = pl.ds(j * 16, 16)
                    rows[p][r, sl] = rows[p][r, sl] * wt

    def step(c, p, guard):
        rb = (p - 1) % NBUF
        if guard == "static":
            wait_scatter(rb)
            launch(c + NBUF - 1, rb)
        elif guard == "traced":
            @pl.when(c < N_CHUNKS - (NBUF - 1))
            def _():
                wait_scatter(rb)
                launch(c + NBUF - 1, rb)
        wait_gather(p)
        scale(c, p)
        wait_dst(p)
        launch_scatter(c, p)

    for c in range(NBUF - 1):
        launch(c, c)

    launch(NBUF - 1, NBUF - 1)
    wait_gather(0)
    scale(0, 0)
    wait_dst(0)
    launch_scatter(0, 0)

    for c in range(1, NBUF - 1):
        step(c, c, "static")

    n_main = N_CHUNKS - (NBUF - 1)
    n_quads = n_main // NBUF
    base_p = (NBUF - 1) % NBUF

    @pl.loop(0, n_quads)
    def _quad(q):
        c0 = (NBUF - 1) + q * NBUF
        for k in range(NBUF):
            step(c0 + k, (base_p + k) % NBUF, "traced")

    for c in range((NBUF - 1) + n_quads * NBUF, N_CHUNKS):
        step(c, c % NBUF, "traced-tail")

    for c in range(N_CHUNKS - NBUF, N_CHUNKS):
        wait_scatter(c % NBUF)

    plsc.subcore_barrier()

    r0 = sid * ROWS_PER_TILE
    pltpu.sync_copy(acc.at[pl.ds(r0, ROWS_PER_TILE)],
                    out_hbm.at[pl.ds(cid * N_PAD + r0, ROWS_PER_TILE)])


_sc_edges = pl.kernel(
    _sc_body,
    out_type=jax.ShapeDtypeStruct((NC * N_PAD, DIM), jnp.float32),
    mesh=plsc.VectorSubcoreMesh(core_axis_name="c", subcore_axis_name="s"),
    scratch_types=(
        [pltpu.VMEM_SHARED((N_PAD, DIM), jnp.float32),
         pltpu.VMEM((E_PER_W,), jnp.int32),
         pltpu.VMEM((E_PER_W,), jnp.float32)]
        + [pltpu.VMEM((CHUNK,), jnp.int32) for _ in range(NBUF)]
        + [pltpu.VMEM((CHUNK, DIM), jnp.float32) for _ in range(NBUF)]
        + [pltpu.SemaphoreType.DMA for _ in range(3 * NBUF)]
    ),
)


def kernel(node_emb, edges, edge_weight, W):
    dst = edges[0].astype(jnp.int32)
    src = edges[1].astype(jnp.int32)
    partials = _sc_edges(node_emb, src, dst, edge_weight)
    return _sum_matmul(partials, W)[:N_NODES]
